# baseline (speedup 1.0000x reference)
import jax
import jax.numpy as jnp
from jax.experimental import pallas as pl
from jax.experimental.pallas import tpu as pltpu

NUM_CLASSES = 1000
N_PROPS = 128
ROW_BLOCK = 512


def _body(x_ref, o_ref):
    x = x_ref[...]
    ids = x[:, 0:1].astype(jnp.int32)
    cols = jax.lax.broadcasted_iota(jnp.int32, (ROW_BLOCK, NUM_CLASSES), 1)
    o_ref[:, :NUM_CLASSES] = (cols == ids).astype(jnp.float32)
    o_ref[:, NUM_CLASSES:] = x[:, 1:]


def kernel(X):
    B, F = X.shape
    out_shape = jax.ShapeDtypeStruct((B, NUM_CLASSES + N_PROPS), X.dtype)
    grid = (B // ROW_BLOCK,)
    return pl.pallas_call(
        _body,
        grid=grid,
        in_specs=[pl.BlockSpec((ROW_BLOCK, F), lambda i: (i, 0))],
        out_specs=pl.BlockSpec((ROW_BLOCK, NUM_CLASSES + N_PROPS), lambda i: (i, 0)),
        out_shape=out_shape,
    )(X)
